# Initial kernel scaffold; baseline (speedup 1.0000x reference)
import jax
import jax.numpy as jnp
from jax.experimental import pallas as pl


def kernel(x, attention_mask, Wk, Wq, Wv, Wu, bu, Wh):
    raise NotImplementedError("write your pallas kernel here")


# speedup vs baseline: 117.1440x; 117.1440x over previous
import functools

import jax
import jax.numpy as jnp
from jax import lax
from jax.experimental import pallas as pl
from jax.experimental.pallas import tpu as pltpu
from jax.experimental.pallas import tpu_sc as plsc

_EMB = 1024
_H = 16
_HS = 16
_K = 4
_P = 8
_NC = 2
_NS = 16
_CB1 = 512
_NCH = 512
_CB3 = 1024


def _proj_body(x_ref, w_ref, means_ref, invs_ref, vals_ref, kt_ref, qt_ref,
               vt_ref):
    xb = x_ref[0]
    y = jnp.dot(xb, w_ref[...], preferred_element_type=jnp.float32)
    h0 = y[:, 0:64]
    h1 = y[:, 64:128]
    h2 = y[:, 128:192]
    means = jax.nn.sigmoid(h0) * 4095.0
    sp = jnp.maximum(h1, 0.0) + jnp.log(1.0 + jnp.exp(-jnp.abs(h1)))
    invs = 1.0 / (sp + 1e-2)
    means_ref[0] = means.T
    invs_ref[0] = invs.T
    vals_ref[0] = h2.T
    kt_ref[0] = y[:, 192:448].T * 0.25
    qt_ref[0] = y[:, 448:704].T * 0.25
    vt_ref[0] = y[:, 704:960].T.astype(jnp.bfloat16)


def _sc_body(means_hbm, invs_hbm, vals_hbm, q_hbm, k_hbm, vp_hbm, out_hbm,
             kc, vw, qc, mch, ich, vch, acc, inv_s, ob):
    C = 4096
    ci = lax.axis_index("c")
    si = lax.axis_index("s")
    wid = si * _NC + ci
    b = wid // _H
    h = wid % _H
    lanes = lax.iota(jnp.int32, 16)

    pltpu.sync_copy(k_hbm.at[b, pl.ds(h * (16 * C), 16 * C)], kc)
    pltpu.sync_copy(vp_hbm.at[b, pl.ds(h * (16 * (C // 2)), 16 * (C // 2))], vw)

    def _idx_dens(j):
        ms = [mch[km, pl.ds(j * 16, 16)] for km in range(_K)]
        ivs = [ich[km, pl.ds(j * 16, 16)] for km in range(_K)]
        idxs = []
        for km in range(_K):
            fl = ms[km].astype(jnp.int32)
            idxs.append(jnp.minimum(fl, C - 1))
            idxs.append(jnp.minimum(fl + 1, C - 1))
        dups = [None] * _P
        for p in range(1, _P):
            d = idxs[p] == idxs[0]
            for p2 in range(1, p):
                d = jnp.logical_or(d, idxs[p] == idxs[p2])
            dups[p] = d
        dens = []
        for p in range(_P):
            fp = idxs[p].astype(jnp.float32)
            row = []
            for km in range(_K):
                t = (fp - ms[km]) * ivs[km]
                dd = jnp.exp(-0.5 * (t * t))
                if p > 0:
                    dd = jnp.where(dups[p], 0.0, dd)
                row.append(dd)
            dens.append(row)
        return idxs, dens

    for r in range(_P * _K):
        acc[pl.ds(r * 16, 16)] = jnp.zeros((16,), jnp.float32)

    def _pass_a(ch, _):
        c0 = ch * _NCH
        pltpu.sync_copy(means_hbm.at[b, pl.ds(h * 4, 4), pl.ds(c0, _NCH)], mch)
        pltpu.sync_copy(invs_hbm.at[b, pl.ds(h * 4, 4), pl.ds(c0, _NCH)], ich)

        def inner(j, _):
            _, dens = _idx_dens(j)
            for p in range(_P):
                for km in range(_K):
                    r = p * _K + km
                    acc[pl.ds(r * 16, 16)] = (
                        acc[pl.ds(r * 16, 16)] + dens[p][km])
            return 0

        lax.fori_loop(0, _NCH // 16, inner, 0)
        return 0

    lax.fori_loop(0, C // _NCH, _pass_a, 0)

    for r in range(_P * _K):
        tot = None
        for j in range(16):
            g = plsc.load_gather(
                acc, [jnp.full((16,), r * 16 + j, jnp.int32)])
            tot = g if tot is None else tot + g
        inv_s[r, :] = 1.0 / (tot + 1e-8)

    def _pass_b(ch, _):
        c0 = ch * _NCH
        pltpu.sync_copy(means_hbm.at[b, pl.ds(h * 4, 4), pl.ds(c0, _NCH)], mch)
        pltpu.sync_copy(invs_hbm.at[b, pl.ds(h * 4, 4), pl.ds(c0, _NCH)], ich)
        pltpu.sync_copy(vals_hbm.at[b, pl.ds(h * 4, 4), pl.ds(c0, _NCH)], vch)
        pltpu.sync_copy(q_hbm.at[b, pl.ds(h * 16, 16), pl.ds(c0, _NCH)], qc)

        def inner(j, _):
            idxs, dens = _idx_dens(j)
            vls = [vch[km, pl.ds(j * 16, 16)] for km in range(_K)]
            ws = []
            for p in range(_P):
                w = None
                for km in range(_K):
                    term = vls[km] * dens[p][km] * inv_s[p * _K + km, :]
                    w = term if w is None else w + term
                ws.append(w)
            dots = []
            for p in range(_P):
                d = None
                for s in range(_HS):
                    qs = qc[s, pl.ds(j * 16, 16)]
                    kg = plsc.load_gather(
                        kc, [idxs[p] + jnp.int32(s * 4096)])
                    term = qs * kg
                    d = term if d is None else d + term
                dots.append(d)
            ts = [ws[p] * dots[p] for p in range(_P)]
            m = ts[0]
            for p in range(1, _P):
                m = jnp.maximum(m, ts[p])
            es = [jnp.exp(ts[p] - m) for p in range(_P)]
            se = es[0]
            for p in range(1, _P):
                se = se + es[p]
            inv = 1.0 / se
            nws = [es[p] * inv for p in range(_P)]
            halfs = [idxs[p] >> 1 for p in range(_P)]
            odds = [(idxs[p] & 1) == 1 for p in range(_P)]
            row_base = j * 256 + lanes * 16
            for s in range(_HS):
                accv = None
                for p in range(_P):
                    g = plsc.load_gather(
                        vw, [halfs[p] + jnp.int32(s * 2048)])
                    bits = jnp.where(odds[p], g & jnp.int32(-65536), g << 16)
                    vf = lax.bitcast_convert_type(bits, jnp.float32)
                    term = nws[p] * vf
                    accv = term if accv is None else accv + term
                plsc.store_scatter(ob, [row_base + s], accv)
            return 0

        lax.fori_loop(0, _NCH // 16, inner, 0)
        pltpu.sync_copy(ob, out_hbm.at[b, h, pl.ds(c0 * 16, _NCH * 16)])
        return 0

    lax.fori_loop(0, C // _NCH, _pass_b, 0)


def _out_body(hr_ref, wu_ref, bu_ref, o_ref):
    o_ref[0] = (jnp.dot(hr_ref[0], wu_ref[...],
                        preferred_element_type=jnp.float32) + bu_ref[...])


def kernel(x, attention_mask, Wk, Wq, Wv, Wu, bu, Wh):
    del attention_mask
    B, C, E = x.shape
    wh_r = Wh.reshape(E, _H, _K, 3)
    w_all = jnp.concatenate(
        [wh_r[..., 0].reshape(E, _H * _K), wh_r[..., 1].reshape(E, _H * _K),
         wh_r[..., 2].reshape(E, _H * _K), Wk, Wq, Wv], axis=1)

    f32 = jnp.float32
    means_t, invs_t, vals_t, k_t, q_t, v_t = pl.pallas_call(
        _proj_body,
        grid=(B, C // _CB1),
        in_specs=[
            pl.BlockSpec((1, _CB1, E), lambda b, c: (b, c, 0)),
            pl.BlockSpec((E, 960), lambda b, c: (0, 0)),
        ],
        out_specs=[
            pl.BlockSpec((1, 64, _CB1), lambda b, c: (b, 0, c)),
            pl.BlockSpec((1, 64, _CB1), lambda b, c: (b, 0, c)),
            pl.BlockSpec((1, 64, _CB1), lambda b, c: (b, 0, c)),
            pl.BlockSpec((1, 256, _CB1), lambda b, c: (b, 0, c)),
            pl.BlockSpec((1, 256, _CB1), lambda b, c: (b, 0, c)),
            pl.BlockSpec((1, 256, _CB1), lambda b, c: (b, 0, c)),
        ],
        out_shape=[
            jax.ShapeDtypeStruct((B, 64, C), f32),
            jax.ShapeDtypeStruct((B, 64, C), f32),
            jax.ShapeDtypeStruct((B, 64, C), f32),
            jax.ShapeDtypeStruct((B, 256, C), f32),
            jax.ShapeDtypeStruct((B, 256, C), f32),
            jax.ShapeDtypeStruct((B, 256, C), jnp.bfloat16),
        ],
    )(x, w_all)

    vp = lax.bitcast_convert_type(
        v_t.reshape(B, 256, C // 2, 2), jnp.int32)

    mesh = plsc.VectorSubcoreMesh(
        core_axis_name="c", subcore_axis_name="s",
        num_cores=_NC, num_subcores=_NS)
    hr = pl.kernel(
        _sc_body,
        out_type=jax.ShapeDtypeStruct((B, _H, C * _HS), f32),
        mesh=mesh,
        compiler_params=pltpu.CompilerParams(needs_layout_passes=False),
        scratch_types=[
            pltpu.VMEM((16 * C,), f32),
            pltpu.VMEM((16 * (C // 2),), jnp.int32),
            pltpu.VMEM((16, _NCH), f32),
            pltpu.VMEM((4, _NCH), f32),
            pltpu.VMEM((4, _NCH), f32),
            pltpu.VMEM((4, _NCH), f32),
            pltpu.VMEM((_P * _K * 16,), f32),
            pltpu.VMEM((_P * _K, 16), f32),
            pltpu.VMEM((_NCH * 16,), f32),
        ],
    )(means_t, invs_t, vals_t, q_t,
      k_t.reshape(B, 256 * C), vp.reshape(B, 256 * (C // 2)))

    united = hr.reshape(B, C, _H * _HS)
    out = pl.pallas_call(
        _out_body,
        grid=(B, C // _CB3),
        in_specs=[
            pl.BlockSpec((1, _CB3, _H * _HS), lambda b, c: (b, c, 0)),
            pl.BlockSpec((_H * _HS, E), lambda b, c: (0, 0)),
            pl.BlockSpec((1, E), lambda b, c: (0, 0)),
        ],
        out_specs=pl.BlockSpec((1, _CB3, E), lambda b, c: (b, c, 0)),
        out_shape=jax.ShapeDtypeStruct((B, C, E), f32),
    )(united, Wu, bu.reshape(1, E))
    return out


# speedup vs baseline: 136.9874x; 1.1694x over previous
import functools

import jax
import jax.numpy as jnp
from jax import lax
from jax.experimental import pallas as pl
from jax.experimental.pallas import tpu as pltpu
from jax.experimental.pallas import tpu_sc as plsc

_EMB = 1024
_H = 16
_HS = 16
_K = 4
_P = 8
_NC = 2
_NS = 16
_CB1 = 512
_NCH = 512
_CB3 = 1024


def _proj_body(x_ref, wh_ref, wkqv_ref, means_ref, invs_ref, vals_ref,
               kt_ref, qt_ref, vt_ref):
    xb = x_ref[0]
    y = jnp.dot(xb, wh_ref[...], preferred_element_type=jnp.float32)
    h0 = y[:, 0:64]
    h1 = y[:, 64:128]
    h2 = y[:, 128:192]
    means = jax.nn.sigmoid(h0) * 4095.0
    sp = jnp.maximum(h1, 0.0) + jnp.log(1.0 + jnp.exp(-jnp.abs(h1)))
    invs = 1.0 / (sp + 1e-2)
    means_ref[0] = means.T
    invs_ref[0] = invs.T
    vals_ref[0] = h2.T
    y2 = jnp.dot(xb.astype(jnp.bfloat16), wkqv_ref[...],
                 preferred_element_type=jnp.float32)
    kt_ref[0] = y2[:, 0:256].T * 0.25
    qt_ref[0] = y2[:, 256:512].T * 0.25
    vt_ref[0] = y2[:, 512:768].T.astype(jnp.bfloat16)


def _sc_body(means_hbm, invs_hbm, vals_hbm, q_hbm, k_hbm, vp_hbm, out_hbm,
             kc, vw, qc, mch, ich, vch, acc, inv_s, ob):
    C = 4096
    ci = lax.axis_index("c")
    si = lax.axis_index("s")
    wid = si * _NC + ci
    b = wid // _H
    h = wid % _H
    lanes = lax.iota(jnp.int32, 16)

    pltpu.sync_copy(k_hbm.at[b, pl.ds(h * (16 * C), 16 * C)], kc)
    pltpu.sync_copy(vp_hbm.at[b, pl.ds(h * (8 * C), 8 * C)], vw)

    def _idx_dens(j):
        ms = [mch[km, pl.ds(j * 16, 16)] for km in range(_K)]
        ivs = [ich[km, pl.ds(j * 16, 16)] for km in range(_K)]
        idxs = []
        for km in range(_K):
            fl = ms[km].astype(jnp.int32)
            idxs.append(jnp.minimum(fl, C - 1))
            idxs.append(jnp.minimum(fl + 1, C - 1))
        dups = [None] * _P
        for p in range(1, _P):
            d = idxs[p] == idxs[0]
            for p2 in range(1, p):
                d = jnp.logical_or(d, idxs[p] == idxs[p2])
            dups[p] = d
        dens = []
        for p in range(_P):
            fp = idxs[p].astype(jnp.float32)
            row = []
            for km in range(_K):
                t = (fp - ms[km]) * ivs[km]
                dd = jnp.exp(-0.5 * (t * t))
                if p > 0:
                    dd = jnp.where(dups[p], 0.0, dd)
                row.append(dd)
            dens.append(row)
        return idxs, dens

    for r in range(_P * _K):
        acc[pl.ds(r * 16, 16)] = jnp.zeros((16,), jnp.float32)

    def _pass_a(ch, _):
        c0 = ch * _NCH
        pltpu.sync_copy(means_hbm.at[b, pl.ds(h * 4, 4), pl.ds(c0, _NCH)], mch)
        pltpu.sync_copy(invs_hbm.at[b, pl.ds(h * 4, 4), pl.ds(c0, _NCH)], ich)

        def inner(j, _):
            _, dens = _idx_dens(j)
            for p in range(_P):
                for km in range(_K):
                    r = p * _K + km
                    acc[pl.ds(r * 16, 16)] = (
                        acc[pl.ds(r * 16, 16)] + dens[p][km])
            return 0

        lax.fori_loop(0, _NCH // 16, inner, 0)
        return 0

    lax.fori_loop(0, C // _NCH, _pass_a, 0)

    for r in range(_P * _K):
        tot = None
        for j in range(16):
            g = plsc.load_gather(
                acc, [jnp.full((16,), r * 16 + j, jnp.int32)])
            tot = g if tot is None else tot + g
        inv_s[r, :] = 1.0 / (tot + 1e-8)

    def _pass_b(ch, _):
        c0 = ch * _NCH
        pltpu.sync_copy(means_hbm.at[b, pl.ds(h * 4, 4), pl.ds(c0, _NCH)], mch)
        pltpu.sync_copy(invs_hbm.at[b, pl.ds(h * 4, 4), pl.ds(c0, _NCH)], ich)
        pltpu.sync_copy(vals_hbm.at[b, pl.ds(h * 4, 4), pl.ds(c0, _NCH)], vch)
        pltpu.sync_copy(q_hbm.at[b, pl.ds(h * 16, 16), pl.ds(c0, _NCH)], qc)

        def inner(j, _):
            idxs, dens = _idx_dens(j)
            vls = [vch[km, pl.ds(j * 16, 16)] for km in range(_K)]
            ws = []
            for p in range(_P):
                w = None
                for km in range(_K):
                    term = vls[km] * dens[p][km] * inv_s[p * _K + km, :]
                    w = term if w is None else w + term
                ws.append(w)
            dots = []
            for p in range(_P):
                d = None
                for s in range(_HS):
                    qs = qc[s, pl.ds(j * 16, 16)]
                    kg = plsc.load_gather(
                        kc, [idxs[p] + jnp.int32(s * 4096)])
                    term = qs * kg
                    d = term if d is None else d + term
                dots.append(d)
            ts = [ws[p] * dots[p] for p in range(_P)]
            m = ts[0]
            for p in range(1, _P):
                m = jnp.maximum(m, ts[p])
            es = [jnp.exp(ts[p] - m) for p in range(_P)]
            se = es[0]
            for p in range(1, _P):
                se = se + es[p]
            inv = 1.0 / se
            nws = [es[p] * inv for p in range(_P)]
            row_base = j * 256 + lanes * 16
            for sp in range(_HS // 2):
                alo = None
                ahi = None
                for p in range(_P):
                    g = plsc.load_gather(
                        vw, [idxs[p] + jnp.int32(sp * 4096)])
                    vlo = lax.bitcast_convert_type(g << 16, jnp.float32)
                    vhi = lax.bitcast_convert_type(
                        g & jnp.int32(-65536), jnp.float32)
                    tlo = nws[p] * vlo
                    thi = nws[p] * vhi
                    alo = tlo if alo is None else alo + tlo
                    ahi = thi if ahi is None else ahi + thi
                plsc.store_scatter(ob, [row_base + (2 * sp)], alo)
                plsc.store_scatter(ob, [row_base + (2 * sp + 1)], ahi)
            return 0

        lax.fori_loop(0, _NCH // 16, inner, 0)
        pltpu.sync_copy(ob, out_hbm.at[b, h, pl.ds(c0 * 16, _NCH * 16)])
        return 0

    lax.fori_loop(0, C // _NCH, _pass_b, 0)


def _out_body(hr_ref, wu_ref, bu_ref, o_ref):
    o_ref[0] = (jnp.dot(hr_ref[0].astype(jnp.bfloat16), wu_ref[...],
                        preferred_element_type=jnp.float32) + bu_ref[...])


def kernel(x, attention_mask, Wk, Wq, Wv, Wu, bu, Wh):
    del attention_mask
    B, C, E = x.shape
    wh_r = Wh.reshape(E, _H, _K, 3)
    w_h = jnp.concatenate(
        [wh_r[..., 0].reshape(E, _H * _K), wh_r[..., 1].reshape(E, _H * _K),
         wh_r[..., 2].reshape(E, _H * _K)], axis=1)
    w_kqv = jnp.concatenate([Wk, Wq, Wv], axis=1).astype(jnp.bfloat16)

    f32 = jnp.float32
    means_t, invs_t, vals_t, k_t, q_t, v_t = pl.pallas_call(
        _proj_body,
        grid=(B, C // _CB1),
        in_specs=[
            pl.BlockSpec((1, _CB1, E), lambda b, c: (b, c, 0)),
            pl.BlockSpec((E, 192), lambda b, c: (0, 0)),
            pl.BlockSpec((E, 768), lambda b, c: (0, 0)),
        ],
        out_specs=[
            pl.BlockSpec((1, 64, _CB1), lambda b, c: (b, 0, c)),
            pl.BlockSpec((1, 64, _CB1), lambda b, c: (b, 0, c)),
            pl.BlockSpec((1, 64, _CB1), lambda b, c: (b, 0, c)),
            pl.BlockSpec((1, 256, _CB1), lambda b, c: (b, 0, c)),
            pl.BlockSpec((1, 256, _CB1), lambda b, c: (b, 0, c)),
            pl.BlockSpec((1, 256, _CB1), lambda b, c: (b, 0, c)),
        ],
        out_shape=[
            jax.ShapeDtypeStruct((B, 64, C), f32),
            jax.ShapeDtypeStruct((B, 64, C), f32),
            jax.ShapeDtypeStruct((B, 64, C), f32),
            jax.ShapeDtypeStruct((B, 256, C), f32),
            jax.ShapeDtypeStruct((B, 256, C), f32),
            jax.ShapeDtypeStruct((B, 256, C), jnp.bfloat16),
        ],
    )(x, w_h, w_kqv)

    v16 = lax.bitcast_convert_type(v_t, jnp.uint16)
    lo = v16[:, 0::2, :].astype(jnp.uint32)
    hi = v16[:, 1::2, :].astype(jnp.uint32)
    vp = lax.bitcast_convert_type(lo | (hi << 16), jnp.int32)

    mesh = plsc.VectorSubcoreMesh(
        core_axis_name="c", subcore_axis_name="s",
        num_cores=_NC, num_subcores=_NS)
    hr = pl.kernel(
        _sc_body,
        out_type=jax.ShapeDtypeStruct((B, _H, C * _HS), f32),
        mesh=mesh,
        compiler_params=pltpu.CompilerParams(needs_layout_passes=False),
        scratch_types=[
            pltpu.VMEM((16 * C,), f32),
            pltpu.VMEM((8 * C,), jnp.int32),
            pltpu.VMEM((16, _NCH), f32),
            pltpu.VMEM((4, _NCH), f32),
            pltpu.VMEM((4, _NCH), f32),
            pltpu.VMEM((4, _NCH), f32),
            pltpu.VMEM((_P * _K * 16,), f32),
            pltpu.VMEM((_P * _K, 16), f32),
            pltpu.VMEM((_NCH * 16,), f32),
        ],
    )(means_t, invs_t, vals_t, q_t,
      k_t.reshape(B, 256 * C), vp.reshape(B, 128 * C))

    united = hr.reshape(B, C, _H * _HS)
    out = pl.pallas_call(
        _out_body,
        grid=(B, C // _CB3),
        in_specs=[
            pl.BlockSpec((1, _CB3, _H * _HS), lambda b, c: (b, c, 0)),
            pl.BlockSpec((_H * _HS, E), lambda b, c: (0, 0)),
            pl.BlockSpec((1, E), lambda b, c: (0, 0)),
        ],
        out_specs=pl.BlockSpec((1, _CB3, E), lambda b, c: (b, c, 0)),
        out_shape=jax.ShapeDtypeStruct((B, C, E), f32),
    )(united, Wu.astype(jnp.bfloat16), bu.reshape(1, E))
    return out
